# baseline (speedup 1.0000x reference)
import functools
import jax
import jax.numpy as jnp
from jax import lax
from jax.experimental import pallas as pl
from jax.experimental.pallas import tpu as pltpu
from jax.experimental.pallas import tpu_sc as plsc

_NC, _NS, _L = 1, 16, 16
_NW = _NC * _NS


def _sc_body(chunk, n, g,
             x_hbm, b_hbm, a_hbm, tab_hbm, out_hbm,
             x_v, b_v, a_v, tab_v, o_v, sem0, sem1, semo):
    wid = lax.axis_index("s") * _NC + lax.axis_index("c")
    base = lax.min(wid * chunk, n - chunk)
    half = chunk // 2

    def fire_inputs(off, size, sem):
        return [
            pltpu.async_copy(x_hbm.at[pl.ds(base + off, size)],
                             x_v.at[pl.ds(off, size)], sem),
            pltpu.async_copy(b_hbm.at[pl.ds(base + off, size)],
                             b_v.at[pl.ds(off, size)], sem),
            pltpu.async_copy(a_hbm.at[pl.ds(base + off, size)],
                             a_v.at[pl.ds(off, size)], sem),
        ]

    def compute(lo, hi):
        @plsc.parallel_loop(lo, hi, step=_L, unroll=16)
        def _(off):
            sl = pl.ds(off, _L)
            m = plsc.load_gather(tab_v, [b_v[sl]]).astype(jnp.int32)
            c = m * 16 + a_v[sl] + g
            sc = plsc.load_gather(tab_v, [c])
            sh = plsc.load_gather(tab_v, [c + 64])
            o_v[sl] = x_v[sl] * sc + sh

    cps0 = fire_inputs(0, half, sem0)
    cps0.append(pltpu.async_copy(tab_hbm, tab_v, sem0))
    cps1 = fire_inputs(half, half, sem1)
    for c in cps0:
        c.wait()
    compute(0, half)
    st0 = pltpu.async_copy(o_v.at[pl.ds(0, half)],
                           out_hbm.at[pl.ds(base, half)], semo)
    for c in cps1:
        c.wait()
    compute(half, chunk)
    st1 = pltpu.async_copy(o_v.at[pl.ds(half, half)],
                           out_hbm.at[pl.ds(base + half, half)], semo)
    st0.wait()
    st1.wait()


@jax.jit
def kernel(scaled_atomic_energy, batch, modal_type, atom_type, shift, scale):
    n = scaled_atomic_energy.shape[0]
    g = modal_type.shape[0]
    x = scaled_atomic_energy.reshape(-1).astype(jnp.float32)
    b = batch.astype(jnp.int32)
    a = atom_type.astype(jnp.int32)
    tab = jnp.concatenate([
        modal_type.astype(jnp.float32),
        scale.reshape(-1).astype(jnp.float32),
        shift.reshape(-1).astype(jnp.float32),
    ])

    chunk = -(-n // (_NW * _L * 8)) * (_L * 8)
    assert n % 8 == 0 and (_NW - 1) * chunk <= n and chunk <= n

    body = functools.partial(_sc_body, chunk, n, g)
    out = pl.kernel(
        body,
        out_type=jax.ShapeDtypeStruct((n,), jnp.float32),
        mesh=plsc.VectorSubcoreMesh(core_axis_name="c", subcore_axis_name="s",
                                    num_cores=_NC, num_subcores=_NS),
        scratch_types=[
            pltpu.VMEM((chunk,), jnp.float32),
            pltpu.VMEM((chunk,), jnp.int32),
            pltpu.VMEM((chunk,), jnp.int32),
            pltpu.VMEM((tab.shape[0],), jnp.float32),
            pltpu.VMEM((chunk,), jnp.float32),
            pltpu.SemaphoreType.DMA,
            pltpu.SemaphoreType.DMA,
            pltpu.SemaphoreType.DMA,
        ],
        compiler_params=pltpu.CompilerParams(needs_layout_passes=False),
    )(x, b, a, tab)
    return out.reshape(-1, 1)


# speedup vs baseline: 1.0048x; 1.0048x over previous
import functools
import jax
import jax.numpy as jnp
from jax import lax
from jax.experimental import pallas as pl
from jax.experimental.pallas import tpu as pltpu
from jax.experimental.pallas import tpu_sc as plsc

_NC, _NS, _L = 1, 16, 16
_NW = _NC * _NS


def _sc_body(chunk, n, g,
             x_hbm, b_hbm, a_hbm, tab_hbm, out_hbm,
             x_v, b_v, a_v, tab_v, o_v, sem0, sem1, semo):
    wid = lax.axis_index("s") * _NC + lax.axis_index("c")
    base = lax.min(wid * chunk, n - chunk)
    half = chunk // 2

    def fire_inputs(off, size, sem):
        return [
            pltpu.async_copy(x_hbm.at[pl.ds(base + off, size)],
                             x_v.at[pl.ds(off, size)], sem),
            pltpu.async_copy(b_hbm.at[pl.ds(base + off, size)],
                             b_v.at[pl.ds(off, size)], sem),
            pltpu.async_copy(a_hbm.at[pl.ds(base + off, size)],
                             a_v.at[pl.ds(off, size)], sem),
        ]

    def compute(lo, hi):
        @plsc.parallel_loop(lo, hi, step=_L, unroll=8)
        def _(off):
            sl = pl.ds(off, _L)
            m = plsc.load_gather(tab_v, [b_v[sl]]).astype(jnp.int32)
            c = m * 16 + a_v[sl] + g
            sc = plsc.load_gather(tab_v, [c])
            sh = plsc.load_gather(tab_v, [c + 64])
            o_v[sl] = x_v[sl] * sc + sh

    cps0 = fire_inputs(0, half, sem0)
    cps0.append(pltpu.async_copy(tab_hbm, tab_v, sem0))
    cps1 = fire_inputs(half, half, sem1)
    for c in cps0:
        c.wait()
    compute(0, half)
    st0 = pltpu.async_copy(o_v.at[pl.ds(0, half)],
                           out_hbm.at[pl.ds(base, half)], semo)
    for c in cps1:
        c.wait()
    compute(half, chunk)
    st1 = pltpu.async_copy(o_v.at[pl.ds(half, half)],
                           out_hbm.at[pl.ds(base + half, half)], semo)
    st0.wait()
    st1.wait()


@jax.jit
def kernel(scaled_atomic_energy, batch, modal_type, atom_type, shift, scale):
    n = scaled_atomic_energy.shape[0]
    g = modal_type.shape[0]
    x = scaled_atomic_energy.reshape(-1).astype(jnp.float32)
    b = batch.astype(jnp.int32)
    a = atom_type.astype(jnp.int32)
    tab = jnp.concatenate([
        modal_type.astype(jnp.float32),
        scale.reshape(-1).astype(jnp.float32),
        shift.reshape(-1).astype(jnp.float32),
    ])

    chunk = -(-n // (_NW * _L * 8)) * (_L * 8)
    assert n % 8 == 0 and (_NW - 1) * chunk <= n and chunk <= n

    body = functools.partial(_sc_body, chunk, n, g)
    out = pl.kernel(
        body,
        out_type=jax.ShapeDtypeStruct((n,), jnp.float32),
        mesh=plsc.VectorSubcoreMesh(core_axis_name="c", subcore_axis_name="s",
                                    num_cores=_NC, num_subcores=_NS),
        scratch_types=[
            pltpu.VMEM((chunk,), jnp.float32),
            pltpu.VMEM((chunk,), jnp.int32),
            pltpu.VMEM((chunk,), jnp.int32),
            pltpu.VMEM((tab.shape[0],), jnp.float32),
            pltpu.VMEM((chunk,), jnp.float32),
            pltpu.SemaphoreType.DMA,
            pltpu.SemaphoreType.DMA,
            pltpu.SemaphoreType.DMA,
        ],
        compiler_params=pltpu.CompilerParams(needs_layout_passes=False),
    )(x, b, a, tab)
    return out.reshape(-1, 1)
